# baseline (speedup 1.0000x reference)
import functools

import jax
import jax.numpy as jnp
from jax import lax
from jax.experimental import pallas as pl
from jax.experimental.pallas import tpu as pltpu
from jax.experimental.pallas import tpu_sc as plsc

H = 2048
W = 2048
NCH = 10
NOUT = 8
LANES = 16

_info = plsc.get_sparse_core_info()
NC = _info.num_cores
NS = _info.num_subcores
NW = NC * NS
ROWS_PER_W = H // NW

CW = W // 2
NCHUNK = ROWS_PER_W * 2
NBUF = 4


def _magnitude(dx, dy):
    s = dx * dx + dy * dy
    bits = lax.bitcast_convert_type(s, jnp.int32)
    seed = jnp.int32(0x5F3759DF) - (bits >> 1)
    y = lax.bitcast_convert_type(seed, jnp.float32)
    hs = s * jnp.float32(0.5)
    for _ in range(2):
        y = y * (jnp.float32(1.5) - hs * y * y)
    return s * y


def _compute_chunk(in_v, out_v):
    @plsc.parallel_loop(0, CW // LANES, unroll=2)
    def vec_body(i):
        sl = pl.ds(i * LANES, LANES)
        c = [in_v[ch, sl] for ch in range(NOUT)]
        m01 = jnp.maximum(c[0], c[1])
        m23 = jnp.maximum(c[2], c[3])
        m45 = jnp.maximum(c[4], c[5])
        m67 = jnp.maximum(c[6], c[7])
        m = jnp.maximum(jnp.maximum(m01, m23), jnp.maximum(m45, m67))
        mag = _magnitude(in_v[8, sl], in_v[9, sl])
        zero = jnp.zeros((LANES,), jnp.float32)
        for ch in range(NOUT):
            out_v[ch, sl] = jnp.where(c[ch] == m, mag, zero)


def _sc_kernel(x_hbm, out_hbm, *refs):
    ins = refs[0:NBUF]
    outs = refs[NBUF : 2 * NBUF]
    sems_i = refs[2 * NBUF : 3 * NBUF]
    sems_o = refs[3 * NBUF : 4 * NBUF]

    wid = lax.axis_index("s") * NC + lax.axis_index("c")
    chunk0 = wid * NCHUNK

    def in_cp(q, b):
        r = q // 2
        col = (q % 2) * CW
        return pltpu.make_async_copy(
            x_hbm.at[0, :, r, pl.ds(col, CW)], ins[b], sems_i[b]
        )

    def out_cp(q, b):
        r = q // 2
        col = (q % 2) * CW
        return pltpu.make_async_copy(
            outs[b], out_hbm.at[0, :, r, pl.ds(col, CW)], sems_o[b]
        )

    for b in range(NBUF):
        in_cp(chunk0 + b, b).start()

    for b in range(NBUF):
        q = chunk0 + b
        in_cp(q, b).wait()
        _compute_chunk(ins[b], outs[b])
        out_cp(q, b).start()
        in_cp(q + NBUF, b).start()

    def group_body(g, _):
        q0 = chunk0 + NBUF * g
        for b in range(NBUF):
            q = q0 + b
            in_cp(q, b).wait()
            out_cp(q, b).wait()
            _compute_chunk(ins[b], outs[b])
            out_cp(q, b).start()

            @pl.when(g < NCHUNK // NBUF - 1)
            def _():
                in_cp(q + NBUF, b).start()

        return 0

    lax.fori_loop(1, NCHUNK // NBUF, group_body, 0)

    for b in range(NBUF):
        out_cp(chunk0, b).wait()


@jax.jit
def kernel(x):
    mesh = plsc.VectorSubcoreMesh(core_axis_name="c", subcore_axis_name="s")
    scratch = (
        [pltpu.VMEM((NCH, CW), jnp.float32) for _ in range(NBUF)]
        + [pltpu.VMEM((NOUT, CW), jnp.float32) for _ in range(NBUF)]
        + [pltpu.SemaphoreType.DMA for _ in range(NBUF)]
        + [pltpu.SemaphoreType.DMA for _ in range(NBUF)]
    )
    f = functools.partial(
        pl.kernel,
        mesh=mesh,
        out_type=jax.ShapeDtypeStruct((1, NOUT, H, W), jnp.float32),
        scratch_types=scratch,
    )(_sc_kernel)
    return f(x)
